# initial kernel scaffold (unmeasured)
import jax
import jax.numpy as jnp
from jax import lax
from jax.experimental import pallas as pl
from jax.experimental.pallas import tpu as pltpu

N_DEV = 4
BQ = 2
SQ = 512
SKV = 512
HG = 8
DH = 64
DM = 768
DG = HG * DH


def kernel(x, Wq, K_ext, V_ext, Wo):
    bf16 = jnp.bfloat16
    my = lax.axis_index("i")

    xb = x.astype(bf16)
    Wqb = Wq.astype(bf16)
    Wob = Wo.astype(bf16)

    def regroup(a):
        a = lax.dynamic_slice_in_dim(a, my * BQ, BQ, axis=0)
        a = a.reshape(BQ, SKV, N_DEV, HG, DH).transpose(2, 0, 3, 1, 4)
        order = jnp.mod(my - jnp.arange(N_DEV), N_DEV)
        return jnp.take(a, order, axis=0).astype(bf16)

    Kg = regroup(K_ext)
    Vg = regroup(V_ext)

    def body(x_ref, wq_ref, k_ref, v_ref, wo_ref, out_ref,
             wq_buf, wo_buf, wq_send, wq_recv, wo_send, wo_recv):
        my_pos = lax.axis_index("i")
        left = lax.rem(my_pos + N_DEV - 1, N_DEV)
        right = lax.rem(my_pos + 1, N_DEV)

        barrier = pltpu.get_barrier_semaphore()
        pl.semaphore_signal(barrier, inc=1, device_id=(left,),
                            device_id_type=pl.DeviceIdType.MESH)
        pl.semaphore_signal(barrier, inc=1, device_id=(right,),
                            device_id_type=pl.DeviceIdType.MESH)
        pl.semaphore_wait(barrier, 2)

        qi = lax.broadcasted_iota(jnp.int32, (SQ, SKV), 0) // 64
        kj = lax.broadcasted_iota(jnp.int32, (SQ, SKV), 1) // 64
        mask = (qi == kj) | ((kj % 4) == (qi % 4))

        def compute(slot, wq, wo):
            for b in range(BQ):
                q = jnp.dot(x_ref[b], wq, preferred_element_type=bf16)
                qh = jnp.transpose(q.reshape(SQ, HG, DH), (1, 0, 2))
                kg = k_ref[slot, b]
                vg = v_ref[slot, b]
                s = lax.dot_general(
                    qh, kg, (((2,), (2,)), ((0,), (0,))),
                    preferred_element_type=jnp.float32) * 0.125
                s = jnp.where(mask[None], s, -1e9)
                m = jnp.max(s, axis=-1, keepdims=True)
                w = jnp.exp(s - m)
                w = w / jnp.sum(w, axis=-1, keepdims=True)
                wb = w.astype(bf16)
                ctx = lax.dot_general(
                    wb, vg, (((2,), (1,)), ((0,), (0,))),
                    preferred_element_type=bf16)
                ctx2 = jnp.transpose(ctx, (1, 0, 2)).reshape(SQ, DG)
                part = jnp.dot(ctx2, wo, preferred_element_type=jnp.float32)
                if slot == 0:
                    out_ref[b, :, :] = part
                else:
                    out_ref[b, :, :] = out_ref[b, :, :] + part

        sends = []

        def start_send(t, src_wq, src_wo):
            rq = pltpu.make_async_remote_copy(
                src_ref=src_wq, dst_ref=wq_buf.at[t],
                send_sem=wq_send.at[t], recv_sem=wq_recv.at[t],
                device_id=(right,), device_id_type=pl.DeviceIdType.MESH)
            ro = pltpu.make_async_remote_copy(
                src_ref=src_wo, dst_ref=wo_buf.at[t],
                send_sem=wo_send.at[t], recv_sem=wo_recv.at[t],
                device_id=(right,), device_id_type=pl.DeviceIdType.MESH)
            rq.start()
            ro.start()
            sends.append((rq, ro))

        start_send(0, wq_ref, wo_ref)
        compute(0, wq_ref[...], wo_ref[...])

        for t in range(N_DEV - 1):
            recv_q = pltpu.make_async_remote_copy(
                src_ref=wq_buf.at[t], dst_ref=wq_buf.at[t],
                send_sem=wq_send.at[t], recv_sem=wq_recv.at[t],
                device_id=(left,), device_id_type=pl.DeviceIdType.MESH)
            recv_o = pltpu.make_async_remote_copy(
                src_ref=wo_buf.at[t], dst_ref=wo_buf.at[t],
                send_sem=wo_send.at[t], recv_sem=wo_recv.at[t],
                device_id=(left,), device_id_type=pl.DeviceIdType.MESH)
            recv_q.wait_recv()
            recv_o.wait_recv()
            if t < N_DEV - 2:
                start_send(t + 1, wq_buf.at[t], wo_buf.at[t])
            compute(t + 1, wq_buf[t], wo_buf[t])

        for rq, ro in sends:
            rq.wait_send()
            ro.wait_send()

    out = pl.pallas_call(
        body,
        out_shape=jax.ShapeDtypeStruct((BQ, SQ, DM), jnp.float32),
        in_specs=[pl.BlockSpec(memory_space=pltpu.VMEM)] * 5,
        out_specs=pl.BlockSpec(memory_space=pltpu.VMEM),
        scratch_shapes=[
            pltpu.VMEM((N_DEV - 1, DM, DG), bf16),
            pltpu.VMEM((N_DEV - 1, DG, DM), bf16),
            pltpu.SemaphoreType.DMA((N_DEV - 1,)),
            pltpu.SemaphoreType.DMA((N_DEV - 1,)),
            pltpu.SemaphoreType.DMA((N_DEV - 1,)),
            pltpu.SemaphoreType.DMA((N_DEV - 1,)),
        ],
        compiler_params=pltpu.CompilerParams(collective_id=0),
    )(xb, Wqb, Kg, Vg, Wob)
    return out


# baseline (device time: 101532 ns/iter reference)
import jax
import jax.numpy as jnp
from jax import lax
from jax.experimental import pallas as pl
from jax.experimental.pallas import tpu as pltpu

N_DEV = 4
BQ = 2
SQ = 512
SKV = 512
HG = 8
DH = 64
DM = 768
DG = HG * DH


def kernel(x, Wq, K_ext, V_ext, Wo):
    bf16 = jnp.bfloat16
    my = lax.axis_index("i")

    xb = x.astype(bf16)
    Wqb = Wq.astype(bf16)
    Wob = Wo.astype(bf16)

    def regroup(a):
        a = lax.dynamic_slice_in_dim(a, my * BQ, BQ, axis=0)
        a = a.reshape(BQ, SKV, N_DEV, HG, DH).transpose(2, 0, 3, 1, 4)
        order = jnp.mod(my - jnp.arange(N_DEV), N_DEV)
        return jnp.take(a, order, axis=0).astype(bf16)

    Kg = regroup(K_ext)
    Vg = regroup(V_ext)

    def body(x_ref, wq_ref, k_ref, v_ref, wo_ref, out_ref,
             wq_buf, wo_buf, wq_send, wq_recv, wo_send, wo_recv):
        my_pos = lax.axis_index("i")
        left = lax.rem(my_pos + N_DEV - 1, N_DEV)
        right = lax.rem(my_pos + 1, N_DEV)

        barrier = pltpu.get_barrier_semaphore()
        pl.semaphore_signal(barrier, inc=1, device_id=(left,),
                            device_id_type=pl.DeviceIdType.MESH)
        pl.semaphore_signal(barrier, inc=1, device_id=(right,),
                            device_id_type=pl.DeviceIdType.MESH)
        pl.semaphore_wait(barrier, 2)

        qi = lax.broadcasted_iota(jnp.int32, (SQ, SKV), 0) // 64
        kj = lax.broadcasted_iota(jnp.int32, (SQ, SKV), 1) // 64
        mask = (qi == kj) | ((kj % 4) == (qi % 4))

        def compute(slot, wq, wo):
            for b in range(BQ):
                q = jnp.dot(x_ref[b], wq,
                            preferred_element_type=jnp.float32).astype(bf16)
                qh = jnp.transpose(q.reshape(SQ, HG, DH), (1, 0, 2))
                kg = k_ref[slot, b]
                vg = v_ref[slot, b]
                s = lax.dot_general(
                    qh, kg, (((2,), (2,)), ((0,), (0,))),
                    preferred_element_type=jnp.float32) * 0.125
                s = jnp.where(mask[None], s, -1e9)
                m = jnp.max(s, axis=-1, keepdims=True)
                w = jnp.exp(s - m)
                w = w / jnp.sum(w, axis=-1, keepdims=True)
                wb = w.astype(bf16)
                ctx = lax.dot_general(
                    wb, vg, (((2,), (1,)), ((0,), (0,))),
                    preferred_element_type=jnp.float32).astype(bf16)
                ctx2 = jnp.transpose(ctx, (1, 0, 2)).reshape(SQ, DG)
                part = jnp.dot(ctx2, wo, preferred_element_type=jnp.float32)
                if slot == 0:
                    out_ref[b, :, :] = part
                else:
                    out_ref[b, :, :] = out_ref[b, :, :] + part

        sends = []

        def start_send(t, src_wq, src_wo):
            rq = pltpu.make_async_remote_copy(
                src_ref=src_wq, dst_ref=wq_buf.at[t],
                send_sem=wq_send.at[t], recv_sem=wq_recv.at[t],
                device_id=(right,), device_id_type=pl.DeviceIdType.MESH)
            ro = pltpu.make_async_remote_copy(
                src_ref=src_wo, dst_ref=wo_buf.at[t],
                send_sem=wo_send.at[t], recv_sem=wo_recv.at[t],
                device_id=(right,), device_id_type=pl.DeviceIdType.MESH)
            rq.start()
            ro.start()
            sends.append((rq, ro))

        start_send(0, wq_ref, wo_ref)
        compute(0, wq_ref[...], wo_ref[...])

        for t in range(N_DEV - 1):
            recv_q = pltpu.make_async_remote_copy(
                src_ref=wq_buf.at[t], dst_ref=wq_buf.at[t],
                send_sem=wq_send.at[t], recv_sem=wq_recv.at[t],
                device_id=(left,), device_id_type=pl.DeviceIdType.MESH)
            recv_o = pltpu.make_async_remote_copy(
                src_ref=wo_buf.at[t], dst_ref=wo_buf.at[t],
                send_sem=wo_send.at[t], recv_sem=wo_recv.at[t],
                device_id=(left,), device_id_type=pl.DeviceIdType.MESH)
            recv_q.wait_recv()
            recv_o.wait_recv()
            if t < N_DEV - 2:
                start_send(t + 1, wq_buf.at[t], wo_buf.at[t])
            compute(t + 1, wq_buf[t], wo_buf[t])

        for rq, ro in sends:
            rq.wait_send()
            ro.wait_send()

    out = pl.pallas_call(
        body,
        out_shape=jax.ShapeDtypeStruct((BQ, SQ, DM), jnp.float32),
        in_specs=[pl.BlockSpec(memory_space=pltpu.VMEM)] * 5,
        out_specs=pl.BlockSpec(memory_space=pltpu.VMEM),
        scratch_shapes=[
            pltpu.VMEM((N_DEV - 1, DM, DG), bf16),
            pltpu.VMEM((N_DEV - 1, DG, DM), bf16),
            pltpu.SemaphoreType.DMA((N_DEV - 1,)),
            pltpu.SemaphoreType.DMA((N_DEV - 1,)),
            pltpu.SemaphoreType.DMA((N_DEV - 1,)),
            pltpu.SemaphoreType.DMA((N_DEV - 1,)),
        ],
        compiler_params=pltpu.CompilerParams(collective_id=0),
    )(xb, Wqb, Kg, Vg, Wob)
    return out
